# baseline (speedup 1.0000x reference)
import jax
import jax.numpy as jnp
from jax.experimental import pallas as pl
from jax.experimental.pallas import tpu as pltpu


def _largest_divisor(n, target, mult):
    best = 0
    for d in range(1, min(n, target) + 1):
        if n % d == 0 and d % mult == 0:
            best = d
    if best == 0:
        for d in range(1, min(n, target) + 1):
            if n % d == 0:
                best = d
    return max(best, 1)


def _xw_kernel(x_ref, wl_ref, wh_ref, xwl_ref, xwh_ref):
    x = x_ref[...]
    xwl_ref[...] = jnp.dot(x, wl_ref[...], preferred_element_type=jnp.float32)
    xwh_ref[...] = jnp.dot(x, wh_ref[...], preferred_element_type=jnp.float32)


def _main_kernel(adjl_ref, adjh_ref, xwl_ref, xwh_ref, x_ref, wm_ref,
                 avl_ref, avh_ref, avm_ref, av_ref, out_ref):
    ol = jnp.maximum(
        jnp.dot(adjl_ref[...], xwl_ref[...], preferred_element_type=jnp.float32),
        0.0)
    oh = jnp.maximum(
        jnp.dot(adjh_ref[...], xwh_ref[...], preferred_element_type=jnp.float32),
        0.0)
    om = jnp.maximum(
        jnp.dot(x_ref[...], wm_ref[...], preferred_element_type=jnp.float32),
        0.0)
    f = (jnp.dot(ol, avl_ref[...], preferred_element_type=jnp.float32)
         + jnp.dot(oh, avh_ref[...], preferred_element_type=jnp.float32)
         + jnp.dot(om, avm_ref[...], preferred_element_type=jnp.float32))
    s = jax.nn.sigmoid(f)
    logits = jnp.dot(s, av_ref[...], preferred_element_type=jnp.float32) / 3.0
    a = jax.nn.softmax(logits, axis=1)
    out_ref[...] = 3.0 * (a[:, 0:1] * ol + a[:, 1:2] * oh + a[:, 2:3] * om)


def kernel(x, adj_low, adj_high, weight_low, weight_high, weight_mlp,
           att_vec_low, att_vec_high, att_vec_mlp, att_vec):
    n, d = x.shape
    o = weight_low.shape[1]

    bmx = _largest_divisor(n, 512, 8)
    gx = n // bmx
    xw_low, xw_high = pl.pallas_call(
        _xw_kernel,
        grid=(gx,),
        in_specs=[
            pl.BlockSpec((bmx, d), lambda i: (i, 0)),
            pl.BlockSpec((d, o), lambda i: (0, 0)),
            pl.BlockSpec((d, o), lambda i: (0, 0)),
        ],
        out_specs=[
            pl.BlockSpec((bmx, o), lambda i: (i, 0)),
            pl.BlockSpec((bmx, o), lambda i: (i, 0)),
        ],
        out_shape=[
            jax.ShapeDtypeStruct((n, o), jnp.float32),
            jax.ShapeDtypeStruct((n, o), jnp.float32),
        ],
    )(x, weight_low, weight_high)

    zero = jnp.zeros_like(att_vec_low)
    avl = jnp.concatenate([att_vec_low, zero, zero], axis=1)
    avh = jnp.concatenate([zero, att_vec_high, zero], axis=1)
    avm = jnp.concatenate([zero, zero, att_vec_mlp], axis=1)

    bm = _largest_divisor(n, 200, 8)
    ni = n // bm

    out = pl.pallas_call(
        _main_kernel,
        grid=(ni,),
        in_specs=[
            pl.BlockSpec((bm, n), lambda i: (i, 0)),
            pl.BlockSpec((bm, n), lambda i: (i, 0)),
            pl.BlockSpec((n, o), lambda i: (0, 0)),
            pl.BlockSpec((n, o), lambda i: (0, 0)),
            pl.BlockSpec((bm, d), lambda i: (i, 0)),
            pl.BlockSpec((d, o), lambda i: (0, 0)),
            pl.BlockSpec((o, 3), lambda i: (0, 0)),
            pl.BlockSpec((o, 3), lambda i: (0, 0)),
            pl.BlockSpec((o, 3), lambda i: (0, 0)),
            pl.BlockSpec((3, 3), lambda i: (0, 0)),
        ],
        out_specs=pl.BlockSpec((bm, o), lambda i: (i, 0)),
        out_shape=jax.ShapeDtypeStruct((n, o), jnp.float32),
        compiler_params=pltpu.CompilerParams(
            dimension_semantics=("arbitrary",)),
    )(adj_low, adj_high, xw_low, xw_high, x, weight_mlp, avl, avh, avm, att_vec)
    return out


# speedup vs baseline: 1.0970x; 1.0970x over previous
import jax
import jax.numpy as jnp
from jax.experimental import pallas as pl
from jax.experimental.pallas import tpu as pltpu


def _largest_divisor(n, target, mult):
    best = 0
    for d in range(1, min(n, target) + 1):
        if n % d == 0 and d % mult == 0:
            best = d
    if best == 0:
        for d in range(1, min(n, target) + 1):
            if n % d == 0:
                best = d
    return max(best, 1)


def _main_kernel(bm, adjl_ref, adjh_ref, x_ref, wl_ref, wh_ref, wm_ref,
                 avl_ref, avh_ref, avm_ref, av_ref, out_ref,
                 xwl_ref, xwh_ref):
    i = pl.program_id(0)

    @pl.when(i == 0)
    def _project():
        xfull = x_ref[...]
        xwl_ref[...] = jnp.dot(xfull, wl_ref[...],
                               preferred_element_type=jnp.float32)
        xwh_ref[...] = jnp.dot(xfull, wh_ref[...],
                               preferred_element_type=jnp.float32)

    ol = jnp.maximum(
        jnp.dot(adjl_ref[...], xwl_ref[...], preferred_element_type=jnp.float32),
        0.0)
    oh = jnp.maximum(
        jnp.dot(adjh_ref[...], xwh_ref[...], preferred_element_type=jnp.float32),
        0.0)
    xi = x_ref[pl.ds(i * bm, bm), :]
    om = jnp.maximum(
        jnp.dot(xi, wm_ref[...], preferred_element_type=jnp.float32),
        0.0)
    f = (jnp.dot(ol, avl_ref[...], preferred_element_type=jnp.float32)
         + jnp.dot(oh, avh_ref[...], preferred_element_type=jnp.float32)
         + jnp.dot(om, avm_ref[...], preferred_element_type=jnp.float32))
    s = jax.nn.sigmoid(f)
    logits = jnp.dot(s, av_ref[...], preferred_element_type=jnp.float32) / 3.0
    a = jax.nn.softmax(logits, axis=1)
    out_ref[...] = 3.0 * (a[:, 0:1] * ol + a[:, 1:2] * oh + a[:, 2:3] * om)


def kernel(x, adj_low, adj_high, weight_low, weight_high, weight_mlp,
           att_vec_low, att_vec_high, att_vec_mlp, att_vec):
    import functools

    n, d = x.shape
    o = weight_low.shape[1]

    zero = jnp.zeros_like(att_vec_low)
    avl = jnp.concatenate([att_vec_low, zero, zero], axis=1)
    avh = jnp.concatenate([zero, att_vec_high, zero], axis=1)
    avm = jnp.concatenate([zero, zero, att_vec_mlp], axis=1)

    bm = _largest_divisor(n, 200, 8)
    ni = n // bm

    out = pl.pallas_call(
        functools.partial(_main_kernel, bm),
        grid=(ni,),
        in_specs=[
            pl.BlockSpec((bm, n), lambda i: (i, 0)),
            pl.BlockSpec((bm, n), lambda i: (i, 0)),
            pl.BlockSpec((n, d), lambda i: (0, 0)),
            pl.BlockSpec((d, o), lambda i: (0, 0)),
            pl.BlockSpec((d, o), lambda i: (0, 0)),
            pl.BlockSpec((d, o), lambda i: (0, 0)),
            pl.BlockSpec((o, 3), lambda i: (0, 0)),
            pl.BlockSpec((o, 3), lambda i: (0, 0)),
            pl.BlockSpec((o, 3), lambda i: (0, 0)),
            pl.BlockSpec((3, 3), lambda i: (0, 0)),
        ],
        out_specs=pl.BlockSpec((bm, o), lambda i: (i, 0)),
        out_shape=jax.ShapeDtypeStruct((n, o), jnp.float32),
        scratch_shapes=[
            pltpu.VMEM((n, o), jnp.float32),
            pltpu.VMEM((n, o), jnp.float32),
        ],
        compiler_params=pltpu.CompilerParams(
            dimension_semantics=("arbitrary",)),
    )(adj_low, adj_high, x, weight_low, weight_high, weight_mlp,
      avl, avh, avm, att_vec)
    return out
